# Initial kernel scaffold; baseline (speedup 1.0000x reference)
import jax
import jax.numpy as jnp
from jax.experimental import pallas as pl


def kernel(position_ids, inv_freq):
    raise NotImplementedError("write your pallas kernel here")


# speedup vs baseline: 1.6791x; 1.6791x over previous
import functools

import jax
import jax.numpy as jnp
from jax import lax
from jax.experimental import pallas as pl
from jax.experimental.pallas import tpu as pltpu
from jax.experimental.pallas import tpu_sc as plsc

L = 16
NUM_CORES = 2
NUM_SUBCORES = 16
NW = NUM_CORES * NUM_SUBCORES
S_TOTAL = 2 * 8192
S_PER_W = S_TOTAL // NW
HD2 = 64

_mesh = plsc.VectorSubcoreMesh(core_axis_name="c", subcore_axis_name="s")


@functools.partial(
    pl.kernel,
    mesh=_mesh,
    out_type=jax.ShapeDtypeStruct((S_TOTAL * HD2,), jnp.float32),
    scratch_types=[
        pltpu.VMEM((S_PER_W,), jnp.int32),
        pltpu.VMEM((HD2,), jnp.float32),
        pltpu.VMEM((S_PER_W * HD2,), jnp.float32),
    ],
)
def _rope_theta_sc(pos_hbm, invf_hbm, out_hbm, pos_v, invf_v, out_v):
    wid = lax.axis_index("s") * NUM_CORES + lax.axis_index("c")
    base = wid * S_PER_W
    pltpu.sync_copy(pos_hbm.at[pl.ds(base, S_PER_W)], pos_v)
    pltpu.sync_copy(invf_hbm, invf_v)
    invf = [invf_v[pl.ds(g * L, L)] for g in range(HD2 // L)]

    def body(i, carry):
        row = i * L
        pg = pos_v[pl.ds(row, L)].astype(jnp.float32)
        for j in range(L):
            p = pg[j]
            off = (row + j) * HD2
            for g in range(HD2 // L):
                out_v[pl.ds(off + g * L, L)] = p * invf[g]
        return carry

    lax.fori_loop(0, S_PER_W // L, body, 0)
    pltpu.sync_copy(out_v, out_hbm.at[pl.ds(base * HD2, S_PER_W * HD2)])


def kernel(position_ids, inv_freq):
    pos_flat = position_ids.reshape(-1).astype(jnp.int32)
    out = _rope_theta_sc(pos_flat, inv_freq)
    return out.reshape(position_ids.shape[0], position_ids.shape[1], HD2)


# speedup vs baseline: 2.0166x; 1.2010x over previous
import functools

import jax
import jax.numpy as jnp
from jax import lax
from jax.experimental import pallas as pl
from jax.experimental.pallas import tpu as pltpu
from jax.experimental.pallas import tpu_sc as plsc

L = 16
NUM_CORES = 2
NUM_SUBCORES = 16
NW = NUM_CORES * NUM_SUBCORES
B = 2
S = 8192
W_PER_B = NW // B
S_PER_W = S // W_PER_B
HD2 = 64

_mesh = plsc.VectorSubcoreMesh(core_axis_name="c", subcore_axis_name="s")


@functools.partial(
    pl.kernel,
    mesh=_mesh,
    out_type=jax.ShapeDtypeStruct((B, S, HD2), jnp.float32),
    scratch_types=[
        pltpu.VMEM((S_PER_W,), jnp.int32),
        pltpu.VMEM((HD2,), jnp.float32),
        pltpu.VMEM((S_PER_W, HD2), jnp.float32),
    ],
)
def _rope_theta_sc(pos_hbm, invf_hbm, out_hbm, pos_v, invf_v, out_v):
    wid = lax.axis_index("s") * NUM_CORES + lax.axis_index("c")
    b = wid // W_PER_B
    base = (wid % W_PER_B) * S_PER_W
    pltpu.sync_copy(pos_hbm.at[b, pl.ds(base, S_PER_W)], pos_v)
    pltpu.sync_copy(invf_hbm, invf_v)
    invf = [invf_v[pl.ds(g * L, L)] for g in range(HD2 // L)]

    def body(i, carry):
        row = i * L
        pg = pos_v[pl.ds(row, L)].astype(jnp.float32)
        for j in range(L):
            p = pg[j]
            for g in range(HD2 // L):
                out_v[row + j, pl.ds(g * L, L)] = p * invf[g]
        return carry

    lax.fori_loop(0, S_PER_W // L, body, 0)
    pltpu.sync_copy(out_v, out_hbm.at[b, pl.ds(base, S_PER_W)])


def kernel(position_ids, inv_freq):
    return _rope_theta_sc(position_ids.astype(jnp.int32), inv_freq)
